# Initial kernel scaffold; baseline (speedup 1.0000x reference)
import jax
import jax.numpy as jnp
from jax.experimental import pallas as pl


def kernel(species, edge_src, edge_dst, vec, distances, switch, params):
    raise NotImplementedError("write your pallas kernel here")


# speedup vs baseline: 37.5945x; 37.5945x over previous
import functools

import numpy as np
import jax
import jax.numpy as jnp
from jax import lax
from jax.experimental import pallas as pl
from jax.experimental.pallas import tpu as pltpu
from jax.experimental.pallas import tpu_sc as plsc

N = 10000
E = 320000
DIM = 64
NCH = 16
HID = 64
NBASIS = 8
ZMAX = 64
CUTOFF = 5.0
NLAYERS = 3

NC = 2
NS = 16
NW = NC * NS
EPW = E // NW
G = 80
K = EPW // G
G_SPM = 16
G_M = G * G_SPM
G_NMAC = K // G_SPM
S_SPM = 2
S_M = G * S_SPM
S_NMAC = K // S_SPM
RPS = N // NS

ECOLS = DIM + 3 * NCH + NCH

BE = 3200
_F32 = jnp.float32

_REP3 = np.kron(np.eye(NCH, dtype=np.float32), np.ones((1, 3), np.float32))
_SUM3 = np.ascontiguousarray(_REP3.T)
_TILE16 = np.tile(np.eye(3, dtype=np.float32), (1, NCH))


def _silu(x):
    return x / (1.0 + jnp.exp(-x))


def _worker_id():
    return lax.axis_index("s") * NC + lax.axis_index("c")


@functools.lru_cache(maxsize=None)
def _sc_gather_fn():
    mesh = plsc.VectorSubcoreMesh(core_axis_name="c", subcore_axis_name="s")

    @functools.partial(
        pl.kernel,
        mesh=mesh,
        out_type=[jax.ShapeDtypeStruct((E, DIM), _F32),
                  jax.ShapeDtypeStruct((E, DIM), _F32)],
        scratch_types=[pltpu.VMEM((K, G), jnp.int32),
                       pltpu.VMEM((G_M, DIM), _F32),
                       pltpu.SemaphoreType.DMA],
        compiler_params=pltpu.CompilerParams(use_tc_tiling_on_sc=False),
    )
    def gather_k(ai_hbm, src_hbm, dst_hbm, gs_hbm, gd_hbm, idx_v, rows_v, sem):
        base = _worker_id() * EPW
        for idx_hbm, out_hbm in ((src_hbm, gs_hbm), (dst_hbm, gd_hbm)):
            pltpu.sync_copy(idx_hbm.at[_worker_id()], idx_v)

            def macro(m, _):
                cps = [pltpu.async_copy(ai_hbm.at[idx_v.at[m * G_SPM + i]],
                                        rows_v.at[pl.ds(i * G, G), :], sem)
                       for i in range(G_SPM)]
                for c in cps:
                    c.wait()
                pltpu.sync_copy(rows_v, out_hbm.at[pl.ds(base + m * G_M, G_M), :])
                return 0

            lax.fori_loop(0, G_NMAC, macro, 0)
            ntail = K - G_NMAC * G_SPM
            cps = [pltpu.async_copy(ai_hbm.at[idx_v.at[G_NMAC * G_SPM + i]],
                                    rows_v.at[pl.ds(i * G, G), :], sem)
                   for i in range(ntail)]
            for c in cps:
                c.wait()
            pltpu.sync_copy(rows_v.at[pl.ds(0, ntail * G), :],
                            out_hbm.at[pl.ds(base + G_NMAC * G_M, ntail * G), :])

    return gather_k


def _sc_gather(ai, esrc3, edst3):
    return _sc_gather_fn()(ai, esrc3, edst3)


@functools.lru_cache(maxsize=None)
def _sc_segsum_fn():
    mesh = plsc.VectorSubcoreMesh(core_axis_name="c", subcore_axis_name="s")

    @functools.partial(
        pl.kernel,
        mesh=mesh,
        out_type=jax.ShapeDtypeStruct((2 * N, ECOLS), _F32),
        scratch_types=[pltpu.VMEM((K, G), jnp.int32),
                       pltpu.VMEM((S_M, ECOLS), _F32),
                       pltpu.VMEM_SHARED((N, ECOLS), _F32)],
        compiler_params=pltpu.CompilerParams(use_tc_tiling_on_sc=False),
    )
    def segsum_k(eout_hbm, idx_hbm, zero_hbm, out_hbm, idx_v, vbuf, shared):
        cid = lax.axis_index("c")
        sid = lax.axis_index("s")
        wid = sid * NC + cid
        base = wid * EPW
        pltpu.sync_copy(zero_hbm.at[pl.ds(sid * RPS, RPS), :],
                        shared.at[pl.ds(sid * RPS, RPS), :])
        plsc.subcore_barrier()
        pltpu.sync_copy(idx_hbm.at[wid], idx_v)

        def macro(m, _):
            pltpu.sync_copy(eout_hbm.at[pl.ds(base + m * S_M, S_M), :], vbuf)
            for i in range(S_SPM):
                pltpu.sync_copy(vbuf.at[pl.ds(i * G, G), :],
                                shared.at[idx_v.at[m * S_SPM + i]], add=True)
            return 0

        lax.fori_loop(0, S_NMAC, macro, 0)
        ntail = K - S_NMAC * S_SPM
        if ntail:
            pltpu.sync_copy(eout_hbm.at[pl.ds(base + S_NMAC * S_M, ntail * G), :],
                            vbuf.at[pl.ds(0, ntail * G), :])
            for i in range(ntail):
                pltpu.sync_copy(vbuf.at[pl.ds(i * G, G), :],
                                shared.at[idx_v.at[S_NMAC * S_SPM + i]], add=True)
        plsc.subcore_barrier()
        pltpu.sync_copy(shared.at[pl.ds(sid * RPS, RPS), :],
                        out_hbm.at[pl.ds(cid * N + sid * RPS, RPS), :])

    return segsum_k


def _sc_segsum(eout, esrc3, zeros128):
    return _sc_segsum_fn()(eout, esrc3, zeros128)


def _mlp_in_kernel(x, W0, b0, W1, b1):
    h = _silu(jnp.dot(x, W0, preferred_element_type=_F32) + b0)
    return jnp.dot(h, W1, preferred_element_type=_F32) + b1


def _tc_init(species2, sW, sb, aW0, ab0, aW1, ab1):
    def body(sp_ref, sW_ref, sb_ref, aW0_ref, ab0_ref, aW1_ref, ab1_ref,
             xi_ref, ai_ref):
        sp = sp_ref[...]
        ioz = lax.broadcasted_iota(jnp.int32, (N, ZMAX), 1)
        onehot = (sp == ioz).astype(_F32)
        xi = jnp.dot(onehot, sW_ref[...], preferred_element_type=_F32) + sb_ref[...]
        xi_ref[...] = xi
        ai_ref[...] = _mlp_in_kernel(xi, aW0_ref[...], ab0_ref[...],
                                     aW1_ref[...], ab1_ref[...])

    return pl.pallas_call(
        body,
        out_shape=[jax.ShapeDtypeStruct((N, DIM), _F32),
                   jax.ShapeDtypeStruct((N, DIM), _F32)],
    )(species2, sW, sb, aW0, ab0, aW1, ab1)


def _tc_edge(gs, gd, dist2, sw2, vec, radW, radb, W0c, b0c, W1c, b1c,
             rep3, tile16, has_r):
    hid_w = W0c.shape[1]

    def body(gs_ref, gd_ref, d_ref, sw_ref, v_ref, radW_ref, radb_ref,
             W0_ref, b0_ref, W1_ref, b1_ref, rep3_ref, t16_ref, out_ref):
        d = d_ref[...]
        sw = sw_ref[...]
        nmul = (lax.broadcasted_iota(jnp.int32, (BE, NBASIS), 1) + 1
                ).astype(_F32) * (np.pi / CUTOFF)
        arg = nmul * jnp.minimum(d, np.float32(CUTOFF))
        t = arg * np.float32(0.5 / np.pi)
        k = jnp.floor(t + np.float32(0.5))
        yv = (t - k) * np.float32(2.0 * np.pi)
        y2 = yv * yv
        _c = [np.float32(v) for v in
              (9.9999959983e-01, -1.6666552614e-01, 8.3324028511e-03,
               -1.9808629757e-04, 2.6997106016e-06, -2.0362081410e-08)]
        sn = yv * (_c[0] + y2 * (_c[1] + y2 * (_c[2] + y2 * (
            _c[3] + y2 * (_c[4] + y2 * _c[5])))))
        rb = np.float32(np.sqrt(2.0 / CUTOFF)) * sn / d
        Dij = jnp.dot(rb, radW_ref[...], preferred_element_type=_F32) + radb_ref[...]
        mij = gs_ref[...] * gd_ref[...] * Dij * sw
        out1 = _mlp_in_kernel(mij, W0_ref[...], b0_ref[...], W1_ref[...], b1_ref[...])
        f = out1[:, 0:NCH]
        F = out1[:, 2 * NCH:2 * NCH + 1]
        dirij = v_ref[...] * (sw / d)
        Fij = F * dirij
        fij = jnp.dot(f, rep3_ref[...], preferred_element_type=_F32) * \
            jnp.dot(Fij, t16_ref[...], preferred_element_type=_F32)
        if has_r:
            rw = out1[:, NCH:2 * NCH] * sw
        else:
            rw = jnp.zeros((BE, NCH), _F32)
        out_ref[...] = jnp.concatenate([mij, fij, rw], axis=1)

    nblk = E // BE
    eb = lambda w: pl.BlockSpec((BE, w), lambda i: (i, 0))
    wb = lambda a: pl.BlockSpec(a.shape, lambda i: (0,) * a.ndim)
    return pl.pallas_call(
        body,
        grid=(nblk,),
        in_specs=[eb(DIM), eb(DIM), eb(1), eb(1), eb(3),
                  wb(radW), wb(radb), wb(W0c), wb(b0c), wb(W1c), wb(b1c),
                  wb(rep3), wb(tile16)],
        out_specs=eb(ECOLS),
        out_shape=jax.ShapeDtypeStruct((E, ECOLS), _F32),
    )(gs, gd, dist2, sw2, vec, radW, radb, W0c, b0c, W1c, b1c, rep3, tile16)


def _tc_node(xi, fi, di, seg, RW0, Rb0, RW1, Rb1, uW0, ub0, uW1, ub1,
             reshW, rep3, sum3, first, nxt):
    def body(*refs):
        i = iter(refs)
        xi_ref = next(i)
        fi_ref = None if first else next(i)
        di_ref = None if first else next(i)
        seg_ref = next(i)
        RW0_r, Rb0_r, RW1_r, Rb1_r = next(i), next(i), next(i), next(i)
        uW0_r, ub0_r, uW1_r, ub1_r = next(i), next(i), next(i), next(i)
        reshW_r, rep3_r, sum3_r = next(i), next(i), next(i)
        if nxt is not None:
            aW0_r, ab0_r, aW1_r, ab1_r = next(i), next(i), next(i), next(i)
        xo_ref, fo_ref, do_ref = next(i), next(i), next(i)
        ao_ref = next(i) if nxt is not None else None

        seg = seg_ref[0] + seg_ref[1]
        xi = xi_ref[...] + seg[:, 0:DIM]
        sf = seg[:, DIM:DIM + 3 * NCH]
        fi = sf if first else fi_ref[...] + sf
        rep3 = rep3_r[...]
        phiR = _mlp_in_kernel(xi, RW0_r[...], Rb0_r[...], RW1_r[...], Rb1_r[...])
        deltai = jnp.dot(phiR, rep3, preferred_element_type=_F32) * fi
        if first:
            di = deltai
        else:
            phi_r = seg[:, DIM + 3 * NCH:]
            di = jnp.dot(phi_r, rep3, preferred_element_type=_F32) * di_ref[...] + deltai
        scal = jnp.dot(fi * di, sum3_r[...], preferred_element_type=_F32)
        phiU = _mlp_in_kernel(xi, uW0_r[...], ub0_r[...], uW1_r[...], ub1_r[...])
        dui = jnp.dot(-(phiU * scal), reshW_r[...], preferred_element_type=_F32)
        xi = xi + dui
        xo_ref[...] = xi
        fo_ref[...] = fi
        do_ref[...] = di
        if ao_ref is not None:
            ao_ref[...] = _mlp_in_kernel(xi, aW0_r[...], ab0_r[...],
                                         aW1_r[...], ab1_r[...])

    seg = seg.reshape(2, N, ECOLS)
    ins = [xi] + ([] if first else [fi, di]) + [seg, RW0, Rb0, RW1, Rb1,
                                               uW0, ub0, uW1, ub1, reshW, rep3, sum3]
    outs = [jax.ShapeDtypeStruct((N, DIM), _F32),
            jax.ShapeDtypeStruct((N, 3 * NCH), _F32),
            jax.ShapeDtypeStruct((N, 3 * NCH), _F32)]
    if nxt is not None:
        ins += list(nxt)
        outs.append(jax.ShapeDtypeStruct((N, DIM), _F32))
    BN = 2000
    nb = lambda w: pl.BlockSpec((BN, w), lambda i: (i, 0))
    wb = lambda a: pl.BlockSpec(a.shape, lambda i: (0,) * a.ndim)
    in_specs = ([nb(DIM)] + ([] if first else [nb(3 * NCH), nb(3 * NCH)])
                + [pl.BlockSpec((2, BN, ECOLS), lambda i: (0, i, 0))]
                + [wb(a) for a in ins[(2 if first else 4):]])
    out_specs = [nb(DIM), nb(3 * NCH), nb(3 * NCH)]
    if nxt is not None:
        out_specs.append(nb(DIM))
    return pl.pallas_call(body, grid=(N // BN,), in_specs=in_specs,
                          out_specs=out_specs, out_shape=outs)(*ins)


def kernel(species, edge_src, edge_dst, vec, distances, switch, params):
    species2 = species.reshape(N, 1).astype(jnp.int32)
    esrc3 = edge_src.reshape(NW, K, G).astype(jnp.int32)
    edst3 = edge_dst.reshape(NW, K, G).astype(jnp.int32)
    dist2 = distances.reshape(E, 1).astype(_F32)
    sw2 = switch.reshape(E, 1).astype(_F32)
    vec = vec.astype(_F32)
    zeros128 = jnp.zeros((N, ECOLS), _F32)
    rep3 = jnp.asarray(_REP3)
    sum3 = jnp.asarray(_SUM3)
    tile16 = jnp.asarray(_TILE16)

    def b2(name):
        return params[name].reshape(1, -1).astype(_F32)

    xi, ai = _tc_init(species2, params['species_W'], b2('species_b'),
                      params['l0_phi_a_W0'], b2('l0_phi_a_b0'),
                      params['l0_phi_a_W1'], b2('l0_phi_a_b1'))

    fi = di = None
    for l in range(NLAYERS):
        p = 'l%d_' % l
        has_r = l > 0
        heads = ['phi_f'] + (['phi_r'] if has_r else []) + ['phi_F']
        W0c = jnp.concatenate([params[p + h + '_W0'] for h in heads], axis=1)
        b0c = jnp.concatenate([params[p + h + '_b0'] for h in heads]).reshape(1, -1)
        nh = len(heads)
        W1c = jnp.zeros((nh * HID, 2 * NCH + 1), _F32)
        col0 = {'phi_f': 0, 'phi_r': NCH, 'phi_F': 2 * NCH}
        b1c = jnp.zeros((1, 2 * NCH + 1), _F32)
        for hi, h in enumerate(heads):
            w1 = params[p + h + '_W1']
            W1c = W1c.at[hi * HID:(hi + 1) * HID, col0[h]:col0[h] + w1.shape[1]].set(w1)
            b1c = b1c.at[0, col0[h]:col0[h] + w1.shape[1]].set(params[p + h + '_b1'])

        gs, gd = _sc_gather(ai, esrc3, edst3)
        eout = _tc_edge(gs, gd, dist2, sw2, vec,
                        params[p + 'rad_W'], b2(p + 'rad_b'),
                        W0c, b0c, W1c, b1c, rep3, tile16, has_r)
        seg = _sc_segsum(eout, esrc3, zeros128)

        nxt = None
        if l + 1 < NLAYERS:
            q = 'l%d_' % (l + 1)
            nxt = (params[q + 'phi_a_W0'], b2(q + 'phi_a_b0'),
                   params[q + 'phi_a_W1'], b2(q + 'phi_a_b1'))
        res = _tc_node(xi, fi, di, seg,
                       params[p + 'phi_R_W0'], b2(p + 'phi_R_b0'),
                       params[p + 'phi_R_W1'], b2(p + 'phi_R_b1'),
                       params[p + 'phi_u_W0'], b2(p + 'phi_u_b0'),
                       params[p + 'phi_u_W1'], b2(p + 'phi_u_b1'),
                       params[p + 'reshape_W'], rep3, sum3,
                       first=(l == 0), nxt=nxt)
        if nxt is not None:
            xi, fi, di, ai = res
        else:
            xi, fi, di = res
    return xi


# speedup vs baseline: 39.7194x; 1.0565x over previous
import functools

import numpy as np
import jax
import jax.numpy as jnp
from jax import lax
from jax.experimental import pallas as pl
from jax.experimental.pallas import tpu as pltpu
from jax.experimental.pallas import tpu_sc as plsc

N = 10000
E = 320000
DIM = 64
NCH = 16
HID = 64
NBASIS = 8
ZMAX = 64
CUTOFF = 5.0
NLAYERS = 3

NC = 2
NS = 16
NW = NC * NS
EPW = E // NW
G = 80
K = EPW // G
G_SPM = 8
G_M = G * G_SPM
G_NCH = K // G_SPM
G_TAIL = K - G_NCH * G_SPM
RPS = N // NS

ECOLS = DIM + 3 * NCH + NCH

BE = 3200
_F32 = jnp.float32

_REP3 = np.kron(np.eye(NCH, dtype=np.float32), np.ones((1, 3), np.float32))
_SUM3 = np.ascontiguousarray(_REP3.T)
_TILE16 = np.tile(np.eye(3, dtype=np.float32), (1, NCH))


def _silu(x):
    return x / (1.0 + jnp.exp(-x))


def _worker_id():
    return lax.axis_index("s") * NC + lax.axis_index("c")


@functools.lru_cache(maxsize=None)
def _sc_gather_fn():
    mesh = plsc.VectorSubcoreMesh(core_axis_name="c", subcore_axis_name="s")

    @functools.partial(
        pl.kernel,
        mesh=mesh,
        out_type=[jax.ShapeDtypeStruct((E, DIM), _F32),
                  jax.ShapeDtypeStruct((E, DIM), _F32)],
        scratch_types=[pltpu.VMEM((K, G), jnp.int32),
                       pltpu.VMEM((G_M, DIM), _F32),
                       pltpu.VMEM((G_M, DIM), _F32),
                       pltpu.SemaphoreType.DMA,
                       pltpu.SemaphoreType.DMA],
        compiler_params=pltpu.CompilerParams(use_tc_tiling_on_sc=False),
    )
    def gather_k(ai_hbm, src_hbm, dst_hbm, gs_hbm, gd_hbm,
                 idx_v, buf_a, buf_b, sem_a, sem_b):
        base = _worker_id() * EPW

        def fire(ci, buf, sem):
            for i in range(G_SPM):
                pltpu.async_copy(ai_hbm.at[idx_v.at[ci * G_SPM + i]],
                                 buf.at[pl.ds(i * G, G), :], sem)

        def drain(buf, sem):
            pltpu.make_async_copy(ai_hbm.at[pl.ds(0, G_M), :], buf, sem).wait()

        for idx_hbm, out_hbm in ((src_hbm, gs_hbm), (dst_hbm, gd_hbm)):
            pltpu.sync_copy(idx_hbm.at[_worker_id()], idx_v)

            def wr(ci, buf):
                pltpu.sync_copy(buf, out_hbm.at[pl.ds(base + ci * G_M, G_M), :])

            fire(0, buf_a, sem_a)

            def body(t, _):
                fire(2 * t + 1, buf_b, sem_b)
                drain(buf_a, sem_a)
                wr(2 * t, buf_a)
                fire(2 * t + 2, buf_a, sem_a)
                drain(buf_b, sem_b)
                wr(2 * t + 1, buf_b)
                return 0

            lax.fori_loop(0, (G_NCH - 1) // 2, body, 0)
            drain(buf_a, sem_a)
            wr(G_NCH - 1, buf_a)
            for i in range(G_TAIL):
                pltpu.async_copy(ai_hbm.at[idx_v.at[G_NCH * G_SPM + i]],
                                 buf_b.at[pl.ds(i * G, G), :], sem_b)
            pltpu.make_async_copy(ai_hbm.at[pl.ds(0, G_TAIL * G), :],
                                  buf_b.at[pl.ds(0, G_TAIL * G), :], sem_b).wait()
            pltpu.sync_copy(buf_b.at[pl.ds(0, G_TAIL * G), :],
                            out_hbm.at[pl.ds(base + G_NCH * G_M, G_TAIL * G), :])

    return gather_k


def _sc_gather(ai, esrc3, edst3):
    return _sc_gather_fn()(ai, esrc3, edst3)


@functools.lru_cache(maxsize=None)
def _sc_segsum_fn():
    mesh = plsc.VectorSubcoreMesh(core_axis_name="c", subcore_axis_name="s")

    @functools.partial(
        pl.kernel,
        mesh=mesh,
        out_type=jax.ShapeDtypeStruct((2 * N, ECOLS), _F32),
        scratch_types=[pltpu.VMEM((K, G), jnp.int32),
                       pltpu.VMEM((G, ECOLS), _F32),
                       pltpu.VMEM((G, ECOLS), _F32),
                       pltpu.VMEM_SHARED((N, ECOLS), _F32),
                       pltpu.SemaphoreType.DMA,
                       pltpu.SemaphoreType.DMA],
        compiler_params=pltpu.CompilerParams(use_tc_tiling_on_sc=False),
    )
    def segsum_k(eout_hbm, idx_hbm, zero_hbm, out_hbm,
                 idx_v, buf_a, buf_b, shared, sem_a, sem_b):
        cid = lax.axis_index("c")
        sid = lax.axis_index("s")
        wid = sid * NC + cid
        base = wid * EPW
        pltpu.sync_copy(zero_hbm.at[pl.ds(sid * RPS, RPS), :],
                        shared.at[pl.ds(sid * RPS, RPS), :])
        plsc.subcore_barrier()
        pltpu.sync_copy(idx_hbm.at[wid], idx_v)

        def rd(m, buf, sem):
            pltpu.async_copy(eout_hbm.at[pl.ds(base + m * G, G), :], buf, sem)

        def drain(buf, sem):
            pltpu.make_async_copy(eout_hbm.at[pl.ds(0, G), :], buf, sem).wait()

        def sc_add(m, buf):
            pltpu.sync_copy(buf, shared.at[idx_v.at[m]], add=True)

        rd(0, buf_a, sem_a)

        def body(t, _):
            rd(2 * t + 1, buf_b, sem_b)
            drain(buf_a, sem_a)
            sc_add(2 * t, buf_a)
            rd(2 * t + 2, buf_a, sem_a)
            drain(buf_b, sem_b)
            sc_add(2 * t + 1, buf_b)
            return 0

        lax.fori_loop(0, (K - 1) // 2, body, 0)
        drain(buf_a, sem_a)
        sc_add(K - 1, buf_a)
        plsc.subcore_barrier()
        pltpu.sync_copy(shared.at[pl.ds(sid * RPS, RPS), :],
                        out_hbm.at[pl.ds(cid * N + sid * RPS, RPS), :])

    return segsum_k


def _sc_segsum(eout, esrc3, zeros128):
    return _sc_segsum_fn()(eout, esrc3, zeros128)


def _mlp_in_kernel(x, W0, b0, W1, b1):
    h = _silu(jnp.dot(x, W0, preferred_element_type=_F32) + b0)
    return jnp.dot(h, W1, preferred_element_type=_F32) + b1


def _bdot(x, w):
    return jnp.dot(x.astype(jnp.bfloat16), w.astype(jnp.bfloat16),
                   preferred_element_type=_F32)


def _tc_init(species2, sW, sb, aW0, ab0, aW1, ab1):
    def body(sp_ref, sW_ref, sb_ref, aW0_ref, ab0_ref, aW1_ref, ab1_ref,
             xi_ref, ai_ref):
        sp = sp_ref[...]
        ioz = lax.broadcasted_iota(jnp.int32, (N, ZMAX), 1)
        onehot = (sp == ioz).astype(_F32)
        xi = jnp.dot(onehot, sW_ref[...], preferred_element_type=_F32) + sb_ref[...]
        xi_ref[...] = xi
        ai_ref[...] = _mlp_in_kernel(xi, aW0_ref[...], ab0_ref[...],
                                     aW1_ref[...], ab1_ref[...])

    return pl.pallas_call(
        body,
        out_shape=[jax.ShapeDtypeStruct((N, DIM), _F32),
                   jax.ShapeDtypeStruct((N, DIM), _F32)],
    )(species2, sW, sb, aW0, ab0, aW1, ab1)


def _tc_edge(gs, gd, dist2, sw2, vec, radW, radb, W0c, b0c, W1c, b1c,
             rep3, tile16, has_r):
    hid_w = W0c.shape[1]

    def body(gs_ref, gd_ref, d_ref, sw_ref, v_ref, radW_ref, radb_ref,
             W0_ref, b0_ref, W1_ref, b1_ref, rep3_ref, t16_ref, out_ref):
        d = d_ref[...]
        sw = sw_ref[...]
        nmul = (lax.broadcasted_iota(jnp.int32, (BE, NBASIS), 1) + 1
                ).astype(_F32) * (np.pi / CUTOFF)
        arg = nmul * jnp.minimum(d, np.float32(CUTOFF))
        t = arg * np.float32(0.5 / np.pi)
        k = jnp.floor(t + np.float32(0.5))
        yv = (t - k) * np.float32(2.0 * np.pi)
        y2 = yv * yv
        _c = [np.float32(v) for v in
              (9.9999959983e-01, -1.6666552614e-01, 8.3324028511e-03,
               -1.9808629757e-04, 2.6997106016e-06, -2.0362081410e-08)]
        sn = yv * (_c[0] + y2 * (_c[1] + y2 * (_c[2] + y2 * (
            _c[3] + y2 * (_c[4] + y2 * _c[5])))))
        rb = np.float32(np.sqrt(2.0 / CUTOFF)) * sn / d
        Dij = jnp.dot(rb, radW_ref[...], preferred_element_type=_F32) + radb_ref[...]
        mij = gs_ref[...] * gd_ref[...] * Dij * sw
        h = _silu(_bdot(mij, W0_ref[...]) + b0_ref[...])
        out1 = _bdot(h, W1_ref[...]) + b1_ref[...]
        f = out1[:, 0:NCH]
        F = out1[:, 2 * NCH:2 * NCH + 1]
        dirij = v_ref[...] * (sw / d)
        Fij = F * dirij
        fij = jnp.dot(f, rep3_ref[...], preferred_element_type=_F32) * \
            jnp.dot(Fij, t16_ref[...], preferred_element_type=_F32)
        if has_r:
            rw = out1[:, NCH:2 * NCH] * sw
        else:
            rw = jnp.zeros((BE, NCH), _F32)
        out_ref[...] = jnp.concatenate([mij, fij, rw], axis=1)

    nblk = E // BE
    eb = lambda w: pl.BlockSpec((BE, w), lambda i: (i, 0))
    wb = lambda a: pl.BlockSpec(a.shape, lambda i: (0,) * a.ndim)
    return pl.pallas_call(
        body,
        grid=(nblk,),
        in_specs=[eb(DIM), eb(DIM), eb(1), eb(1), eb(3),
                  wb(radW), wb(radb), wb(W0c), wb(b0c), wb(W1c), wb(b1c),
                  wb(rep3), wb(tile16)],
        out_specs=eb(ECOLS),
        out_shape=jax.ShapeDtypeStruct((E, ECOLS), _F32),
    )(gs, gd, dist2, sw2, vec, radW, radb, W0c, b0c, W1c, b1c, rep3, tile16)


def _tc_node(xi, fi, di, seg, RW0, Rb0, RW1, Rb1, uW0, ub0, uW1, ub1,
             reshW, rep3, sum3, first, nxt):
    def body(*refs):
        i = iter(refs)
        xi_ref = next(i)
        fi_ref = None if first else next(i)
        di_ref = None if first else next(i)
        seg_ref = next(i)
        RW0_r, Rb0_r, RW1_r, Rb1_r = next(i), next(i), next(i), next(i)
        uW0_r, ub0_r, uW1_r, ub1_r = next(i), next(i), next(i), next(i)
        reshW_r, rep3_r, sum3_r = next(i), next(i), next(i)
        if nxt is not None:
            aW0_r, ab0_r, aW1_r, ab1_r = next(i), next(i), next(i), next(i)
        xo_ref, fo_ref, do_ref = next(i), next(i), next(i)
        ao_ref = next(i) if nxt is not None else None

        seg = seg_ref[0] + seg_ref[1]
        xi = xi_ref[...] + seg[:, 0:DIM]
        sf = seg[:, DIM:DIM + 3 * NCH]
        fi = sf if first else fi_ref[...] + sf
        rep3 = rep3_r[...]
        phiR = _mlp_in_kernel(xi, RW0_r[...], Rb0_r[...], RW1_r[...], Rb1_r[...])
        deltai = jnp.dot(phiR, rep3, preferred_element_type=_F32) * fi
        if first:
            di = deltai
        else:
            phi_r = seg[:, DIM + 3 * NCH:]
            di = jnp.dot(phi_r, rep3, preferred_element_type=_F32) * di_ref[...] + deltai
        scal = jnp.dot(fi * di, sum3_r[...], preferred_element_type=_F32)
        phiU = _mlp_in_kernel(xi, uW0_r[...], ub0_r[...], uW1_r[...], ub1_r[...])
        dui = jnp.dot(-(phiU * scal), reshW_r[...], preferred_element_type=_F32)
        xi = xi + dui
        xo_ref[...] = xi
        fo_ref[...] = fi
        do_ref[...] = di
        if ao_ref is not None:
            ao_ref[...] = _mlp_in_kernel(xi, aW0_r[...], ab0_r[...],
                                         aW1_r[...], ab1_r[...])

    seg = seg.reshape(2, N, ECOLS)
    ins = [xi] + ([] if first else [fi, di]) + [seg, RW0, Rb0, RW1, Rb1,
                                               uW0, ub0, uW1, ub1, reshW, rep3, sum3]
    outs = [jax.ShapeDtypeStruct((N, DIM), _F32),
            jax.ShapeDtypeStruct((N, 3 * NCH), _F32),
            jax.ShapeDtypeStruct((N, 3 * NCH), _F32)]
    if nxt is not None:
        ins += list(nxt)
        outs.append(jax.ShapeDtypeStruct((N, DIM), _F32))
    BN = 2000
    nb = lambda w: pl.BlockSpec((BN, w), lambda i: (i, 0))
    wb = lambda a: pl.BlockSpec(a.shape, lambda i: (0,) * a.ndim)
    in_specs = ([nb(DIM)] + ([] if first else [nb(3 * NCH), nb(3 * NCH)])
                + [pl.BlockSpec((2, BN, ECOLS), lambda i: (0, i, 0))]
                + [wb(a) for a in ins[(2 if first else 4):]])
    out_specs = [nb(DIM), nb(3 * NCH), nb(3 * NCH)]
    if nxt is not None:
        out_specs.append(nb(DIM))
    return pl.pallas_call(body, grid=(N // BN,), in_specs=in_specs,
                          out_specs=out_specs, out_shape=outs)(*ins)


def kernel(species, edge_src, edge_dst, vec, distances, switch, params):
    species2 = species.reshape(N, 1).astype(jnp.int32)
    esrc3 = edge_src.reshape(NW, K, G).astype(jnp.int32)
    edst3 = edge_dst.reshape(NW, K, G).astype(jnp.int32)
    dist2 = distances.reshape(E, 1).astype(_F32)
    sw2 = switch.reshape(E, 1).astype(_F32)
    vec = vec.astype(_F32)
    zeros128 = jnp.zeros((N, ECOLS), _F32)
    rep3 = jnp.asarray(_REP3)
    sum3 = jnp.asarray(_SUM3)
    tile16 = jnp.asarray(_TILE16)

    def b2(name):
        return params[name].reshape(1, -1).astype(_F32)

    xi, ai = _tc_init(species2, params['species_W'], b2('species_b'),
                      params['l0_phi_a_W0'], b2('l0_phi_a_b0'),
                      params['l0_phi_a_W1'], b2('l0_phi_a_b1'))

    fi = di = None
    for l in range(NLAYERS):
        p = 'l%d_' % l
        has_r = l > 0
        heads = ['phi_f'] + (['phi_r'] if has_r else []) + ['phi_F']
        W0c = jnp.concatenate([params[p + h + '_W0'] for h in heads], axis=1)
        b0c = jnp.concatenate([params[p + h + '_b0'] for h in heads]).reshape(1, -1)
        nh = len(heads)
        W1c = jnp.zeros((nh * HID, 2 * NCH + 1), _F32)
        col0 = {'phi_f': 0, 'phi_r': NCH, 'phi_F': 2 * NCH}
        b1c = jnp.zeros((1, 2 * NCH + 1), _F32)
        for hi, h in enumerate(heads):
            w1 = params[p + h + '_W1']
            W1c = W1c.at[hi * HID:(hi + 1) * HID, col0[h]:col0[h] + w1.shape[1]].set(w1)
            b1c = b1c.at[0, col0[h]:col0[h] + w1.shape[1]].set(params[p + h + '_b1'])

        gs, gd = _sc_gather(ai, esrc3, edst3)
        eout = _tc_edge(gs, gd, dist2, sw2, vec,
                        params[p + 'rad_W'], b2(p + 'rad_b'),
                        W0c, b0c, W1c, b1c, rep3, tile16, has_r)
        seg = _sc_segsum(eout, esrc3, zeros128)

        nxt = None
        if l + 1 < NLAYERS:
            q = 'l%d_' % (l + 1)
            nxt = (params[q + 'phi_a_W0'], b2(q + 'phi_a_b0'),
                   params[q + 'phi_a_W1'], b2(q + 'phi_a_b1'))
        res = _tc_node(xi, fi, di, seg,
                       params[p + 'phi_R_W0'], b2(p + 'phi_R_b0'),
                       params[p + 'phi_R_W1'], b2(p + 'phi_R_b1'),
                       params[p + 'phi_u_W0'], b2(p + 'phi_u_b0'),
                       params[p + 'phi_u_W1'], b2(p + 'phi_u_b1'),
                       params[p + 'reshape_W'], rep3, sum3,
                       first=(l == 0), nxt=nxt)
        if nxt is not None:
            xi, fi, di, ai = res
        else:
            xi, fi, di = res
    return xi


# speedup vs baseline: 40.3492x; 1.0159x over previous
import functools

import numpy as np
import jax
import jax.numpy as jnp
from jax import lax
from jax.experimental import pallas as pl
from jax.experimental.pallas import tpu as pltpu
from jax.experimental.pallas import tpu_sc as plsc

N = 10000
E = 320000
DIM = 64
NCH = 16
HID = 64
NBASIS = 8
ZMAX = 64
CUTOFF = 5.0
NLAYERS = 3

NC = 2
NS = 16
NW = NC * NS
G = 80
G_SPM = 8
G_M = G * G_SPM
RPS = N // NS
ECA = 163840
ECB = E - ECA

ECOLS = DIM + 3 * NCH + NCH

BE = 2560
_F32 = jnp.float32

_REP3 = np.kron(np.eye(NCH, dtype=np.float32), np.ones((1, 3), np.float32))
_SUM3 = np.ascontiguousarray(_REP3.T)
_TILE16 = np.tile(np.eye(3, dtype=np.float32), (1, NCH))


def _silu(x):
    return x / (1.0 + jnp.exp(-x))


def _worker_id():
    return lax.axis_index("s") * NC + lax.axis_index("c")


@functools.lru_cache(maxsize=None)
def _sc_gather_fn(ec):
    epw = ec // NW
    k = epw // G
    g_nch = k // G_SPM
    g_tail = k - g_nch * G_SPM
    t_loop = (g_nch - 1) // 2
    rem = g_nch - 1 - 2 * t_loop
    mesh = plsc.VectorSubcoreMesh(core_axis_name="c", subcore_axis_name="s")

    @functools.partial(
        pl.kernel,
        mesh=mesh,
        out_type=[jax.ShapeDtypeStruct((ec, DIM), _F32),
                  jax.ShapeDtypeStruct((ec, DIM), _F32)],
        scratch_types=[pltpu.VMEM((k, G), jnp.int32),
                       pltpu.VMEM((G_M, DIM), _F32),
                       pltpu.VMEM((G_M, DIM), _F32),
                       pltpu.SemaphoreType.DMA,
                       pltpu.SemaphoreType.DMA],
        compiler_params=pltpu.CompilerParams(use_tc_tiling_on_sc=False),
    )
    def gather_k(ai_hbm, src_hbm, dst_hbm, gs_hbm, gd_hbm,
                 idx_v, buf_a, buf_b, sem_a, sem_b):
        base = _worker_id() * epw

        def fire(ci, buf, sem):
            for i in range(G_SPM):
                pltpu.async_copy(ai_hbm.at[idx_v.at[ci * G_SPM + i]],
                                 buf.at[pl.ds(i * G, G), :], sem)

        def drain(buf, sem):
            pltpu.make_async_copy(ai_hbm.at[pl.ds(0, G_M), :], buf, sem).wait()

        for idx_hbm, out_hbm in ((src_hbm, gs_hbm), (dst_hbm, gd_hbm)):
            pltpu.sync_copy(idx_hbm.at[_worker_id()], idx_v)

            def wr(ci, buf):
                pltpu.sync_copy(buf, out_hbm.at[pl.ds(base + ci * G_M, G_M), :])

            fire(0, buf_a, sem_a)

            def body(t, _):
                fire(2 * t + 1, buf_b, sem_b)
                drain(buf_a, sem_a)
                wr(2 * t, buf_a)
                fire(2 * t + 2, buf_a, sem_a)
                drain(buf_b, sem_b)
                wr(2 * t + 1, buf_b)
                return 0

            lax.fori_loop(0, t_loop, body, 0)
            drain(buf_a, sem_a)
            wr(2 * t_loop, buf_a)
            if rem:
                fire(g_nch - 1, buf_b, sem_b)
                drain(buf_b, sem_b)
                wr(g_nch - 1, buf_b)
            if g_tail:
                for i in range(g_tail):
                    pltpu.async_copy(ai_hbm.at[idx_v.at[g_nch * G_SPM + i]],
                                     buf_a.at[pl.ds(i * G, G), :], sem_a)
                pltpu.make_async_copy(ai_hbm.at[pl.ds(0, g_tail * G), :],
                                      buf_a.at[pl.ds(0, g_tail * G), :],
                                      sem_a).wait()
                pltpu.sync_copy(buf_a.at[pl.ds(0, g_tail * G), :],
                                out_hbm.at[pl.ds(base + g_nch * G_M,
                                                 g_tail * G), :])

    return gather_k


def _sc_gather(ai, esrc3, edst3, ec):
    return _sc_gather_fn(ec)(ai, esrc3, edst3)


@functools.lru_cache(maxsize=None)
def _sc_segsum_fn(ec):
    epw = ec // NW
    k = epw // G
    t_loop = (k - 1) // 2
    rem = k - 1 - 2 * t_loop
    mesh = plsc.VectorSubcoreMesh(core_axis_name="c", subcore_axis_name="s")

    @functools.partial(
        pl.kernel,
        mesh=mesh,
        out_type=jax.ShapeDtypeStruct((2 * N, ECOLS), _F32),
        scratch_types=[pltpu.VMEM((k, G), jnp.int32),
                       pltpu.VMEM((G, ECOLS), _F32),
                       pltpu.VMEM((G, ECOLS), _F32),
                       pltpu.VMEM_SHARED((N, ECOLS), _F32),
                       pltpu.SemaphoreType.DMA,
                       pltpu.SemaphoreType.DMA],
        compiler_params=pltpu.CompilerParams(use_tc_tiling_on_sc=False),
    )
    def segsum_k(eout_hbm, idx_hbm, zero_hbm, out_hbm,
                 idx_v, buf_a, buf_b, shared, sem_a, sem_b):
        cid = lax.axis_index("c")
        sid = lax.axis_index("s")
        wid = sid * NC + cid
        base = wid * epw
        pltpu.sync_copy(zero_hbm.at[pl.ds(sid * RPS, RPS), :],
                        shared.at[pl.ds(sid * RPS, RPS), :])
        plsc.subcore_barrier()
        pltpu.sync_copy(idx_hbm.at[wid], idx_v)

        def rd(m, buf, sem):
            pltpu.async_copy(eout_hbm.at[pl.ds(base + m * G, G), :], buf, sem)

        def drain(buf, sem):
            pltpu.make_async_copy(eout_hbm.at[pl.ds(0, G), :], buf, sem).wait()

        def sc_add(m, buf):
            pltpu.sync_copy(buf, shared.at[idx_v.at[m]], add=True)

        rd(0, buf_a, sem_a)

        def body(t, _):
            rd(2 * t + 1, buf_b, sem_b)
            drain(buf_a, sem_a)
            sc_add(2 * t, buf_a)
            rd(2 * t + 2, buf_a, sem_a)
            drain(buf_b, sem_b)
            sc_add(2 * t + 1, buf_b)
            return 0

        lax.fori_loop(0, t_loop, body, 0)
        drain(buf_a, sem_a)
        sc_add(2 * t_loop, buf_a)
        if rem:
            rd(k - 1, buf_b, sem_b)
            drain(buf_b, sem_b)
            sc_add(k - 1, buf_b)
        plsc.subcore_barrier()
        pltpu.sync_copy(shared.at[pl.ds(sid * RPS, RPS), :],
                        out_hbm.at[pl.ds(cid * N + sid * RPS, RPS), :])

    return segsum_k


def _sc_segsum(eout, esrc3, zeros128, ec):
    return _sc_segsum_fn(ec)(eout, esrc3, zeros128)


def _mlp_in_kernel(x, W0, b0, W1, b1):
    h = _silu(jnp.dot(x, W0, preferred_element_type=_F32) + b0)
    return jnp.dot(h, W1, preferred_element_type=_F32) + b1


def _bdot(x, w):
    return jnp.dot(x.astype(jnp.bfloat16), w.astype(jnp.bfloat16),
                   preferred_element_type=_F32)


def _tc_init(species2, sW, sb, aW0, ab0, aW1, ab1):
    def body(sp_ref, sW_ref, sb_ref, aW0_ref, ab0_ref, aW1_ref, ab1_ref,
             xi_ref, ai_ref):
        sp = sp_ref[...]
        ioz = lax.broadcasted_iota(jnp.int32, (N, ZMAX), 1)
        onehot = (sp == ioz).astype(_F32)
        xi = jnp.dot(onehot, sW_ref[...], preferred_element_type=_F32) + sb_ref[...]
        xi_ref[...] = xi
        ai_ref[...] = _mlp_in_kernel(xi, aW0_ref[...], ab0_ref[...],
                                     aW1_ref[...], ab1_ref[...])

    return pl.pallas_call(
        body,
        out_shape=[jax.ShapeDtypeStruct((N, DIM), _F32),
                   jax.ShapeDtypeStruct((N, DIM), _F32)],
    )(species2, sW, sb, aW0, ab0, aW1, ab1)


def _tc_edge(gs, gd, dist2, sw2, vec, radW, radb, W0c, b0c, W1c, b1c,
             rep3, tile16, has_r):
    hid_w = W0c.shape[1]

    def body(gs_ref, gd_ref, d_ref, sw_ref, v_ref, radW_ref, radb_ref,
             W0_ref, b0_ref, W1_ref, b1_ref, rep3_ref, t16_ref, out_ref):
        d = d_ref[...]
        sw = sw_ref[...]
        nmul = (lax.broadcasted_iota(jnp.int32, (BE, NBASIS), 1) + 1
                ).astype(_F32) * (np.pi / CUTOFF)
        arg = nmul * jnp.minimum(d, np.float32(CUTOFF))
        t = arg * np.float32(0.5 / np.pi)
        k = jnp.floor(t + np.float32(0.5))
        yv = (t - k) * np.float32(2.0 * np.pi)
        y2 = yv * yv
        _c = [np.float32(v) for v in
              (9.9999959983e-01, -1.6666552614e-01, 8.3324028511e-03,
               -1.9808629757e-04, 2.6997106016e-06, -2.0362081410e-08)]
        sn = yv * (_c[0] + y2 * (_c[1] + y2 * (_c[2] + y2 * (
            _c[3] + y2 * (_c[4] + y2 * _c[5])))))
        rb = np.float32(np.sqrt(2.0 / CUTOFF)) * sn / d
        Dij = jnp.dot(rb, radW_ref[...], preferred_element_type=_F32) + radb_ref[...]
        mij = gs_ref[...] * gd_ref[...] * Dij * sw
        h = _silu(_bdot(mij, W0_ref[...]) + b0_ref[...])
        out1 = _bdot(h, W1_ref[...]) + b1_ref[...]
        f = out1[:, 0:NCH]
        F = out1[:, 2 * NCH:2 * NCH + 1]
        dirij = v_ref[...] * (sw / d)
        Fij = F * dirij
        fij = jnp.dot(f, rep3_ref[...], preferred_element_type=_F32) * \
            jnp.dot(Fij, t16_ref[...], preferred_element_type=_F32)
        if has_r:
            rw = out1[:, NCH:2 * NCH] * sw
        else:
            rw = jnp.zeros((BE, NCH), _F32)
        out_ref[...] = jnp.concatenate([mij, fij, rw], axis=1)

    ec = gs.shape[0]
    nblk = ec // BE
    eb = lambda w: pl.BlockSpec((BE, w), lambda i: (i, 0))
    wb = lambda a: pl.BlockSpec(a.shape, lambda i: (0,) * a.ndim)
    return pl.pallas_call(
        body,
        grid=(nblk,),
        in_specs=[eb(DIM), eb(DIM), eb(1), eb(1), eb(3),
                  wb(radW), wb(radb), wb(W0c), wb(b0c), wb(W1c), wb(b1c),
                  wb(rep3), wb(tile16)],
        out_specs=eb(ECOLS),
        out_shape=jax.ShapeDtypeStruct((ec, ECOLS), _F32),
    )(gs, gd, dist2, sw2, vec, radW, radb, W0c, b0c, W1c, b1c, rep3, tile16)


def _tc_node(xi, fi, di, segA, segB, RW0, Rb0, RW1, Rb1, uW0, ub0, uW1, ub1,
             reshW, rep3, sum3, first, nxt):
    def body(*refs):
        i = iter(refs)
        xi_ref = next(i)
        fi_ref = None if first else next(i)
        di_ref = None if first else next(i)
        segA_ref = next(i)
        segB_ref = next(i)
        RW0_r, Rb0_r, RW1_r, Rb1_r = next(i), next(i), next(i), next(i)
        uW0_r, ub0_r, uW1_r, ub1_r = next(i), next(i), next(i), next(i)
        reshW_r, rep3_r, sum3_r = next(i), next(i), next(i)
        if nxt is not None:
            aW0_r, ab0_r, aW1_r, ab1_r = next(i), next(i), next(i), next(i)
        xo_ref, fo_ref, do_ref = next(i), next(i), next(i)
        ao_ref = next(i) if nxt is not None else None

        seg = (segA_ref[0] + segA_ref[1]) + (segB_ref[0] + segB_ref[1])
        xi = xi_ref[...] + seg[:, 0:DIM]
        sf = seg[:, DIM:DIM + 3 * NCH]
        fi = sf if first else fi_ref[...] + sf
        rep3 = rep3_r[...]
        phiR = _mlp_in_kernel(xi, RW0_r[...], Rb0_r[...], RW1_r[...], Rb1_r[...])
        deltai = jnp.dot(phiR, rep3, preferred_element_type=_F32) * fi
        if first:
            di = deltai
        else:
            phi_r = seg[:, DIM + 3 * NCH:]
            di = jnp.dot(phi_r, rep3, preferred_element_type=_F32) * di_ref[...] + deltai
        scal = jnp.dot(fi * di, sum3_r[...], preferred_element_type=_F32)
        phiU = _mlp_in_kernel(xi, uW0_r[...], ub0_r[...], uW1_r[...], ub1_r[...])
        dui = jnp.dot(-(phiU * scal), reshW_r[...], preferred_element_type=_F32)
        xi = xi + dui
        xo_ref[...] = xi
        fo_ref[...] = fi
        do_ref[...] = di
        if ao_ref is not None:
            ao_ref[...] = _mlp_in_kernel(xi, aW0_r[...], ab0_r[...],
                                         aW1_r[...], ab1_r[...])

    segA = segA.reshape(2, N, ECOLS)
    segB = segB.reshape(2, N, ECOLS)
    ins = [xi] + ([] if first else [fi, di]) + [segA, segB, RW0, Rb0, RW1, Rb1,
                                               uW0, ub0, uW1, ub1, reshW, rep3, sum3]
    outs = [jax.ShapeDtypeStruct((N, DIM), _F32),
            jax.ShapeDtypeStruct((N, 3 * NCH), _F32),
            jax.ShapeDtypeStruct((N, 3 * NCH), _F32)]
    if nxt is not None:
        ins += list(nxt)
        outs.append(jax.ShapeDtypeStruct((N, DIM), _F32))
    BN = 2000
    nb = lambda w: pl.BlockSpec((BN, w), lambda i: (i, 0))
    wb = lambda a: pl.BlockSpec(a.shape, lambda i: (0,) * a.ndim)
    in_specs = ([nb(DIM)] + ([] if first else [nb(3 * NCH), nb(3 * NCH)])
                + [pl.BlockSpec((2, BN, ECOLS), lambda i: (0, i, 0))] * 2
                + [wb(a) for a in ins[(3 if first else 5):]])
    out_specs = [nb(DIM), nb(3 * NCH), nb(3 * NCH)]
    if nxt is not None:
        out_specs.append(nb(DIM))
    return pl.pallas_call(body, grid=(N // BN,), in_specs=in_specs,
                          out_specs=out_specs, out_shape=outs)(*ins)


def kernel(species, edge_src, edge_dst, vec, distances, switch, params):
    species2 = species.reshape(N, 1).astype(jnp.int32)
    chunks = []
    off = 0
    for ec in (ECA, ECB):
        epw = ec // NW
        k = epw // G
        chunks.append(dict(
            ec=ec,
            esrc3=lax.slice_in_dim(edge_src, off, off + ec).reshape(
                NW, k, G).astype(jnp.int32),
            edst3=lax.slice_in_dim(edge_dst, off, off + ec).reshape(
                NW, k, G).astype(jnp.int32),
            dist2=lax.slice_in_dim(distances, off, off + ec).reshape(
                ec, 1).astype(_F32),
            sw2=lax.slice_in_dim(switch, off, off + ec).reshape(
                ec, 1).astype(_F32),
            vec=lax.slice_in_dim(vec, off, off + ec).astype(_F32),
        ))
        off += ec
    zeros128 = jnp.zeros((N, ECOLS), _F32)
    rep3 = jnp.asarray(_REP3)
    sum3 = jnp.asarray(_SUM3)
    tile16 = jnp.asarray(_TILE16)

    def b2(name):
        return params[name].reshape(1, -1).astype(_F32)

    xi, ai = _tc_init(species2, params['species_W'], b2('species_b'),
                      params['l0_phi_a_W0'], b2('l0_phi_a_b0'),
                      params['l0_phi_a_W1'], b2('l0_phi_a_b1'))

    fi = di = None
    for l in range(NLAYERS):
        p = 'l%d_' % l
        has_r = l > 0
        heads = ['phi_f'] + (['phi_r'] if has_r else []) + ['phi_F']
        W0c = jnp.concatenate([params[p + h + '_W0'] for h in heads], axis=1)
        b0c = jnp.concatenate([params[p + h + '_b0'] for h in heads]).reshape(1, -1)
        nh = len(heads)
        W1c = jnp.zeros((nh * HID, 2 * NCH + 1), _F32)
        col0 = {'phi_f': 0, 'phi_r': NCH, 'phi_F': 2 * NCH}
        b1c = jnp.zeros((1, 2 * NCH + 1), _F32)
        for hi, h in enumerate(heads):
            w1 = params[p + h + '_W1']
            W1c = W1c.at[hi * HID:(hi + 1) * HID, col0[h]:col0[h] + w1.shape[1]].set(w1)
            b1c = b1c.at[0, col0[h]:col0[h] + w1.shape[1]].set(params[p + h + '_b1'])

        segs = []
        for ch in chunks:
            gs, gd = _sc_gather(ai, ch['esrc3'], ch['edst3'], ch['ec'])
            eout = _tc_edge(gs, gd, ch['dist2'], ch['sw2'], ch['vec'],
                            params[p + 'rad_W'], b2(p + 'rad_b'),
                            W0c, b0c, W1c, b1c, rep3, tile16, has_r)
            segs.append(_sc_segsum(eout, ch['esrc3'], zeros128, ch['ec']))

        nxt = None
        if l + 1 < NLAYERS:
            q = 'l%d_' % (l + 1)
            nxt = (params[q + 'phi_a_W0'], b2(q + 'phi_a_b0'),
                   params[q + 'phi_a_W1'], b2(q + 'phi_a_b1'))
        res = _tc_node(xi, fi, di, segs[0], segs[1],
                       params[p + 'phi_R_W0'], b2(p + 'phi_R_b0'),
                       params[p + 'phi_R_W1'], b2(p + 'phi_R_b1'),
                       params[p + 'phi_u_W0'], b2(p + 'phi_u_b0'),
                       params[p + 'phi_u_W1'], b2(p + 'phi_u_b1'),
                       params[p + 'reshape_W'], rep3, sum3,
                       first=(l == 0), nxt=nxt)
        if nxt is not None:
            xi, fi, di, ai = res
        else:
            xi, fi, di = res
    return xi
